# Initial kernel scaffold; baseline (speedup 1.0000x reference)
import jax
import jax.numpy as jnp
from jax.experimental import pallas as pl


def kernel(x, Wg_w, Wg_b, Wn_w, Wn_b, W_w, W_b, V_w, V_b, W2_w, W2_b):
    raise NotImplementedError("write your pallas kernel here")


# speedup vs baseline: 1.1880x; 1.1880x over previous
import functools

import jax
import jax.numpy as jnp
from jax.experimental import pallas as pl
from jax.experimental.pallas import tpu as pltpu

M = 1024
DH = 2048
E = 8

_SQRT_HALF = 0.7071067811865476


def _gating_body(x_ref, wg_ref, wgb_ref, wn_ref, wnb_ref, noise_ref,
                 gates_ref, idx_ref, gv_ref, imp_ref, load_ref):
    t = pl.program_id(0)
    x = x_ref[...]
    logits = jax.lax.dot_general(
        x, wg_ref[...], (((1,), (1,)), ((), ())),
        preferred_element_type=jnp.float32) + wgb_ref[...]
    pre = jax.lax.dot_general(
        x, wn_ref[...], (((1,), (1,)), ((), ())),
        preferred_element_type=jnp.float32) + wnb_ref[...]
    noise_scale = jax.nn.softplus(pre)
    h = logits + noise_ref[...] * noise_scale

    tm = h.shape[0]
    iota = jax.lax.broadcasted_iota(jnp.int32, (tm, E), 1)
    neg_inf = jnp.float32(-jnp.inf)

    m1 = jnp.max(h, axis=1, keepdims=True)
    i1 = jnp.min(jnp.where(h == m1, iota, E), axis=1, keepdims=True)
    mask1 = iota == i1
    h2 = jnp.where(mask1, neg_inf, h)
    m2 = jnp.max(h2, axis=1, keepdims=True)
    i2 = jnp.min(jnp.where(h2 == m2, iota, E), axis=1, keepdims=True)
    mask2 = iota == i2
    h3 = jnp.where(mask2, neg_inf, h2)
    m3 = jnp.max(h3, axis=1, keepdims=True)

    e2 = jnp.exp(m2 - m1)
    denom = 1.0 + e2
    g1 = 1.0 / denom
    g2 = e2 / denom
    gates = jnp.where(mask1, g1, 0.0) + jnp.where(mask2, g2, 0.0)
    gates_ref[...] = gates
    idx_ref[...] = jnp.concatenate([i1, i2], axis=1)
    gv_ref[...] = jnp.concatenate([g1, g2], axis=1)

    psi = jnp.where(h > m2, m2, jnp.where(h <= m3, m3, h))
    z = (logits - psi) / noise_scale
    p = 0.5 * (1.0 + jax.lax.erf(z * _SQRT_HALF))

    imp_part = jnp.sum(gates, axis=0, keepdims=True)
    load_part = jnp.sum(p, axis=0, keepdims=True)

    @pl.when(t == 0)
    def _():
        imp_ref[...] = imp_part
        load_ref[...] = load_part

    @pl.when(t != 0)
    def _():
        imp_ref[...] += imp_part
        load_ref[...] += load_part


def _gating(flat, wg_w, wg_b, wn_w, wn_b, noise, tm=1024):
    t_tokens = flat.shape[0]
    nt = t_tokens // tm
    return pl.pallas_call(
        _gating_body,
        grid=(nt,),
        in_specs=[
            pl.BlockSpec((tm, M), lambda t: (t, 0)),
            pl.BlockSpec((E, M), lambda t: (0, 0)),
            pl.BlockSpec((1, E), lambda t: (0, 0)),
            pl.BlockSpec((E, M), lambda t: (0, 0)),
            pl.BlockSpec((1, E), lambda t: (0, 0)),
            pl.BlockSpec((tm, E), lambda t: (t, 0)),
        ],
        out_specs=[
            pl.BlockSpec((tm, E), lambda t: (t, 0)),
            pl.BlockSpec((tm, 2), lambda t: (t, 0)),
            pl.BlockSpec((tm, 2), lambda t: (t, 0)),
            pl.BlockSpec((1, E), lambda t: (0, 0)),
            pl.BlockSpec((1, E), lambda t: (0, 0)),
        ],
        out_shape=[
            jax.ShapeDtypeStruct((t_tokens, E), jnp.float32),
            jax.ShapeDtypeStruct((t_tokens, 2), jnp.int32),
            jax.ShapeDtypeStruct((t_tokens, 2), jnp.float32),
            jax.ShapeDtypeStruct((1, E), jnp.float32),
            jax.ShapeDtypeStruct((1, E), jnp.float32),
        ],
    )(flat, wg_w, wg_b, wn_w, wn_b, noise)


def _dense_expert_body(x_ref, gates_ref, ww_ref, wb_ref, vw_ref, vb_ref,
                       w2w_ref, w2b_ref, out_ref, xv_ref):
    e = pl.program_id(1)
    x = x_ref[...]

    @pl.when(e == 0)
    def _():
        xv = jax.lax.dot_general(
            x, vw_ref[...], (((1,), (1,)), ((), ())),
            preferred_element_type=jnp.float32) + vb_ref[...]
        xv_ref[...] = xv
        out_ref[...] = jnp.zeros_like(out_ref)

    xw = jax.lax.dot_general(
        x, ww_ref[0], (((1,), (1,)), ((), ())),
        preferred_element_type=jnp.float32) + wb_ref[0]
    act = xw * jax.nn.sigmoid(xw) * xv_ref[...]
    contrib = jax.lax.dot_general(
        act.astype(jnp.bfloat16), w2w_ref[...], (((1,), (1,)), ((), ())),
        preferred_element_type=jnp.float32) + w2b_ref[...]
    g = gates_ref[0, 0, :].reshape(-1, 1)
    out_ref[...] += g * contrib


def _dense_experts(flat_bf, gates_t, ww, wb, vw, vb, w2w, w2b, tm=1024):
    t_tokens = flat_bf.shape[0]
    nt = t_tokens // tm
    return pl.pallas_call(
        _dense_expert_body,
        grid=(nt, E),
        in_specs=[
            pl.BlockSpec((tm, M), lambda t, e: (t, 0)),
            pl.BlockSpec((1, 1, tm), lambda t, e: (e, 0, t)),
            pl.BlockSpec((1, DH, M), lambda t, e: (e, 0, 0)),
            pl.BlockSpec((1, 1, DH), lambda t, e: (e, 0, 0)),
            pl.BlockSpec((DH, M), lambda t, e: (0, 0)),
            pl.BlockSpec((1, DH), lambda t, e: (0, 0)),
            pl.BlockSpec((M, DH), lambda t, e: (0, 0)),
            pl.BlockSpec((1, M), lambda t, e: (0, 0)),
        ],
        out_specs=pl.BlockSpec((tm, M), lambda t, e: (t, 0)),
        out_shape=jax.ShapeDtypeStruct((t_tokens, M), jnp.float32),
        scratch_shapes=[pltpu.VMEM((tm, DH), jnp.float32)],
    )(flat_bf, gates_t, ww, wb, vw, vb, w2w, w2b)


def _cv_loss(v):
    return 0.01 * jnp.std(v) / (jnp.mean(v) + 1e-6)


def kernel(x, Wg_w, Wg_b, Wn_w, Wn_b, W_w, W_b, V_w, V_b, W2_w, W2_b):
    B, N, m = x.shape
    flat = x.reshape(B * N, m)
    t_tokens = B * N
    noise = jax.random.normal(jax.random.key(1234), (t_tokens, E),
                              dtype=jnp.float32)

    gates, idx, gv, imp, load = _gating(
        flat, Wg_w, Wg_b.reshape(1, E), Wn_w, Wn_b.reshape(1, E), noise)

    l_moe = _cv_loss(imp[0]) + _cv_loss(load[0])

    flat_out = _dense_experts(
        flat.astype(jnp.bfloat16),
        gates.T.reshape(E, 1, t_tokens),
        W_w.astype(jnp.bfloat16),
        W_b.reshape(E, 1, DH),
        V_w.astype(jnp.bfloat16),
        V_b.reshape(1, DH),
        W2_w.astype(jnp.bfloat16),
        W2_b.reshape(1, M),
    )
    return (flat_out.reshape(B, N, m), l_moe)
